# Initial kernel scaffold; baseline (speedup 1.0000x reference)
import jax
import jax.numpy as jnp
from jax.experimental import pallas as pl


def kernel(src, in_proj_w, in_proj_b, out_proj_w, out_proj_b, norm1_g, norm1_b, norm2_g, norm2_b, router_w, router_b, W1, b1, W2, b2):
    raise NotImplementedError("write your pallas kernel here")


# speedup vs baseline: 1.2631x; 1.2631x over previous
import jax
import jax.numpy as jnp
from jax.experimental import pallas as pl
from jax.experimental.pallas import tpu as pltpu

D = 768
H = 12
DH = 64
DFF = 3072
E = 8
TOPK = 2
BLK = 256
RB = 512


def _attn_body(x_ref, wq_ref, wk_ref, wv_ref, bq_ref, bk_ref, bv_ref, o_ref):
    x = x_ref[...]
    q = jnp.dot(x, wq_ref[0], preferred_element_type=jnp.float32) + bq_ref[0]
    k = jnp.dot(x, wk_ref[0], preferred_element_type=jnp.float32) + bk_ref[0]
    v = jnp.dot(x, wv_ref[0], preferred_element_type=jnp.float32) + bv_ref[0]
    s = jax.lax.dot_general(
        q, k, (((1,), (1,)), ((), ())), preferred_element_type=jnp.float32
    ) * (1.0 / 8.0)
    m = jnp.max(s, axis=1, keepdims=True)
    e = jnp.exp(s - m)
    p = e / jnp.sum(e, axis=1, keepdims=True)
    o_ref[0] = jnp.dot(p, v, preferred_element_type=jnp.float32)


def _post_attn_body(oh_ref, woh_ref, bo_ref, src_ref, g1_ref, b1_ref,
                    wr_ref, br_ref, x_ref, ridx_ref, rgate_ref, psum_ref):
    a = bo_ref[...] + jnp.dot(
        oh_ref[0], woh_ref[0], preferred_element_type=jnp.float32)
    for h in range(1, H):
        a = a + jnp.dot(
            oh_ref[h], woh_ref[h], preferred_element_type=jnp.float32)
    x = src_ref[...] + a
    m = jnp.mean(x, axis=1, keepdims=True)
    v = jnp.mean((x - m) ** 2, axis=1, keepdims=True)
    xn = (x - m) * jax.lax.rsqrt(v + 1e-5) * g1_ref[...] + b1_ref[...]
    x_ref[...] = xn
    logits = (
        jnp.dot(xn, wr_ref[...], preferred_element_type=jnp.float32)
        + br_ref[...]
    )
    lm = jnp.max(logits, axis=1, keepdims=True)
    ex = jnp.exp(logits - lm)
    p = ex / jnp.sum(ex, axis=1, keepdims=True)
    lanes = jax.lax.broadcasted_iota(jnp.int32, p.shape, 1)
    v1 = jnp.max(p, axis=1, keepdims=True)
    i1 = jnp.min(jnp.where(p >= v1, lanes, E), axis=1, keepdims=True)
    p2 = jnp.where(lanes == i1, -1.0, p)
    v2 = jnp.max(p2, axis=1, keepdims=True)
    i2 = jnp.min(jnp.where(p2 >= v2, lanes, E), axis=1, keepdims=True)
    ssum = v1 + v2
    ridx_ref[...] = (
        i1 * (lanes == 0).astype(jnp.int32)
        + i2 * (lanes == 1).astype(jnp.int32)
    )
    rgate_ref[...] = (
        jnp.where(lanes == 0, v1 / ssum, 0.0)
        + jnp.where(lanes == 1, v2 / ssum, 0.0)
    )

    @pl.when(pl.program_id(0) == 0)
    def _():
        psum_ref[...] = jnp.zeros_like(psum_ref)

    psum_ref[...] += jnp.sum(p, axis=0, keepdims=True)


def _moe_body(be_ref, tok_ref, xf_ref, w1_ref, bb1_ref, w2_ref, bb2_ref,
              out_ref, xg_ref):
    i = pl.program_id(0)

    def gath(r, carry):
        t = tok_ref[i * BLK + r]
        xg_ref[pl.ds(r, 1), :] = xf_ref[pl.ds(t, 1), :]
        return carry

    jax.lax.fori_loop(0, BLK, gath, 0, unroll=8)
    h = jnp.maximum(
        jnp.dot(xg_ref[...], w1_ref[0], preferred_element_type=jnp.float32)
        + bb1_ref[0],
        0.0,
    )
    out_ref[...] = (
        jnp.dot(h, w2_ref[0], preferred_element_type=jnp.float32)
        + bb2_ref[0]
    )


def _combine_body(p0_ref, p1_ref, x_ref, outg_ref, gate_ref, g2_ref, b2_ref,
                  y_ref, t0_ref, t1_ref):
    i = pl.program_id(0)

    def gath(r, carry):
        a = p0_ref[i * RB + r]
        b = p1_ref[i * RB + r]
        t0_ref[pl.ds(r, 1), :] = outg_ref[pl.ds(a, 1), :]
        t1_ref[pl.ds(r, 1), :] = outg_ref[pl.ds(b, 1), :]
        return carry

    jax.lax.fori_loop(0, RB, gath, 0, unroll=8)
    g1c = gate_ref[:, 0:1]
    g2c = gate_ref[:, 1:2]
    x = x_ref[...] + g1c * t0_ref[...] + g2c * t1_ref[...]
    m = jnp.mean(x, axis=1, keepdims=True)
    v = jnp.mean((x - m) ** 2, axis=1, keepdims=True)
    y_ref[...] = (x - m) * jax.lax.rsqrt(v + 1e-5) * g2_ref[...] + b2_ref[...]


def kernel(src, in_proj_w, in_proj_b, out_proj_w, out_proj_b,
           norm1_g, norm1_b, norm2_g, norm2_b,
           router_w, router_b, W1, b1, W2, b2):
    Bq, S, d = src.shape
    N = Bq * S
    xflat = src.reshape(N, d)
    f32 = jnp.float32

    wq = in_proj_w[:d].reshape(H, DH, d).transpose(0, 2, 1)
    wk = in_proj_w[d:2 * d].reshape(H, DH, d).transpose(0, 2, 1)
    wv = in_proj_w[2 * d:].reshape(H, DH, d).transpose(0, 2, 1)
    bq = in_proj_b[:d].reshape(H, 1, DH)
    bk = in_proj_b[d:2 * d].reshape(H, 1, DH)
    bv = in_proj_b[2 * d:].reshape(H, 1, DH)

    oh = pl.pallas_call(
        _attn_body,
        grid=(Bq, H),
        in_specs=[
            pl.BlockSpec((S, d), lambda b, h: (b, 0)),
            pl.BlockSpec((1, d, DH), lambda b, h: (h, 0, 0)),
            pl.BlockSpec((1, d, DH), lambda b, h: (h, 0, 0)),
            pl.BlockSpec((1, d, DH), lambda b, h: (h, 0, 0)),
            pl.BlockSpec((1, 1, DH), lambda b, h: (h, 0, 0)),
            pl.BlockSpec((1, 1, DH), lambda b, h: (h, 0, 0)),
            pl.BlockSpec((1, 1, DH), lambda b, h: (h, 0, 0)),
        ],
        out_specs=pl.BlockSpec((1, S, DH), lambda b, h: (h, b, 0)),
        out_shape=jax.ShapeDtypeStruct((H, N, DH), f32),
    )(xflat, wq, wk, wv, bq, bk, bv)

    woh = out_proj_w.T.reshape(H, DH, d)
    x, ridx, rgate, psum = pl.pallas_call(
        _post_attn_body,
        grid=(N // RB,),
        in_specs=[
            pl.BlockSpec((H, RB, DH), lambda i: (0, i, 0)),
            pl.BlockSpec((H, DH, d), lambda i: (0, 0, 0)),
            pl.BlockSpec((1, d), lambda i: (0, 0)),
            pl.BlockSpec((RB, d), lambda i: (i, 0)),
            pl.BlockSpec((1, d), lambda i: (0, 0)),
            pl.BlockSpec((1, d), lambda i: (0, 0)),
            pl.BlockSpec((d, E), lambda i: (0, 0)),
            pl.BlockSpec((1, E), lambda i: (0, 0)),
        ],
        out_specs=[
            pl.BlockSpec((RB, d), lambda i: (i, 0)),
            pl.BlockSpec((RB, E), lambda i: (i, 0)),
            pl.BlockSpec((RB, E), lambda i: (i, 0)),
            pl.BlockSpec((1, E), lambda i: (0, 0)),
        ],
        out_shape=[
            jax.ShapeDtypeStruct((N, d), f32),
            jax.ShapeDtypeStruct((N, E), jnp.int32),
            jax.ShapeDtypeStruct((N, E), f32),
            jax.ShapeDtypeStruct((1, E), f32),
        ],
    )(oh, woh, out_proj_b.reshape(1, d), xflat,
      norm1_g.reshape(1, d), norm1_b.reshape(1, d),
      router_w.T, router_b.reshape(1, E))

    idx = ridx[:, :TOPK]
    e_flat = idx.reshape(-1)
    oh_disp = (
        e_flat[:, None] == jnp.arange(E, dtype=jnp.int32)[None, :]
    ).astype(jnp.int32)
    ranks = jnp.cumsum(oh_disp, axis=0) - oh_disp
    my_rank = jnp.sum(ranks * oh_disp, axis=1)
    counts = jnp.sum(oh_disp, axis=0)
    padded = ((counts + BLK - 1) // BLK) * BLK
    ends = jnp.cumsum(padded)
    starts = ends - padded
    pos = starts[e_flat] + my_rank
    P = TOPK * N + E * BLK
    nb = P // BLK
    tok_pad = jnp.zeros((P,), jnp.int32).at[pos].set(
        jnp.arange(TOPK * N, dtype=jnp.int32) // TOPK)
    blk_e = jnp.minimum(
        jnp.searchsorted(ends, jnp.arange(nb, dtype=jnp.int32) * BLK,
                         side="right").astype(jnp.int32), E - 1)
    pos2 = pos.reshape(N, TOPK)

    grid_spec = pltpu.PrefetchScalarGridSpec(
        num_scalar_prefetch=2,
        grid=(nb,),
        in_specs=[
            pl.BlockSpec((N, d), lambda i, be, tk: (0, 0)),
            pl.BlockSpec((1, d, DFF), lambda i, be, tk: (be[i], 0, 0)),
            pl.BlockSpec((1, 1, DFF), lambda i, be, tk: (be[i], 0, 0)),
            pl.BlockSpec((1, DFF, d), lambda i, be, tk: (be[i], 0, 0)),
            pl.BlockSpec((1, 1, d), lambda i, be, tk: (be[i], 0, 0)),
        ],
        out_specs=pl.BlockSpec((BLK, d), lambda i, be, tk: (i, 0)),
        scratch_shapes=[pltpu.VMEM((BLK, d), f32)],
    )
    outg = pl.pallas_call(
        _moe_body,
        grid_spec=grid_spec,
        out_shape=jax.ShapeDtypeStruct((P, d), f32),
    )(blk_e, tok_pad, x, W1, b1.reshape(E, 1, DFF), W2, b2.reshape(E, 1, d))

    grid_spec2 = pltpu.PrefetchScalarGridSpec(
        num_scalar_prefetch=2,
        grid=(N // RB,),
        in_specs=[
            pl.BlockSpec((RB, d), lambda i, a, b: (i, 0)),
            pl.BlockSpec((P, d), lambda i, a, b: (0, 0)),
            pl.BlockSpec((RB, E), lambda i, a, b: (i, 0)),
            pl.BlockSpec((1, d), lambda i, a, b: (0, 0)),
            pl.BlockSpec((1, d), lambda i, a, b: (0, 0)),
        ],
        out_specs=pl.BlockSpec((RB, d), lambda i, a, b: (i, 0)),
        scratch_shapes=[pltpu.VMEM((RB, d), f32), pltpu.VMEM((RB, d), f32)],
    )
    y = pl.pallas_call(
        _combine_body,
        grid_spec=grid_spec2,
        out_shape=jax.ShapeDtypeStruct((N, d), f32),
    )(pos2[:, 0], pos2[:, 1], x, outg, rgate,
      norm2_g.reshape(1, d), norm2_b.reshape(1, d))

    Nf = jnp.float32(N)
    lb_loss = E * jnp.sum(
        (counts.astype(f32) / Nf) * (psum[0] / Nf))
    return y.reshape(Bq, S, d), lb_loss
